# baseline (speedup 1.0000x reference)
import jax
import jax.numpy as jnp
import numpy as np
from jax.experimental import pallas as pl

SH_C0 = 0.28209479177387814
SH_C1 = 0.4886025119029199
SH_C2 = [1.0925484305920792, -1.0925484305920792, 0.31539156525252005,
         -1.0925484305920792, 0.5462742152960396]

NUM_CODES = 4096
CODE_DIM = 32
NUM_NEIGHBORS = 16
SH_BASIS_DIM = 9
SH_WIDTH = CODE_DIM * SH_BASIS_DIM
NUM_POINTS = 8192
QBLK = 256
IDX_MASK = ~4095
INT_MAX = 2147483647


def _tc_body(qT_ref, vT_ref, cpos_ref, codesT_ref, shT_ref, sel_ref,
             out_c_ref, out_s_ref):
    qT = qT_ref[...]
    cpos = cpos_ref[...]
    mm = jnp.dot(cpos.astype(jnp.bfloat16), qT.astype(jnp.bfloat16),
                 preferred_element_type=jnp.float32)
    qq = jnp.sum(qT * qT, axis=0, keepdims=True)
    cc = jnp.sum(cpos * cpos, axis=1, keepdims=True)
    d2sel = qq + cc - 2.0 * mm
    iota = jax.lax.broadcasted_iota(jnp.int32, (NUM_CODES, QBLK), 0)
    p = jnp.bitwise_or(
        jnp.bitwise_and(jax.lax.bitcast_convert_type(d2sel, jnp.int32),
                        IDX_MASK),
        iota)

    def masked_min(m):
        a = p[:NUM_CODES // 2]
        b = p[NUM_CODES // 2:]
        x = jnp.minimum(jnp.where(a > m, a, INT_MAX),
                        jnp.where(b > m, b, INT_MAX))
        n = NUM_CODES // 2
        while n > 8:
            n //= 2
            x = jnp.minimum(x[:n], x[n:])
        return jnp.min(x, axis=0, keepdims=True)

    def first_min():
        x = jnp.minimum(p[:NUM_CODES // 2], p[NUM_CODES // 2:])
        n = NUM_CODES // 2
        while n > 8:
            n //= 2
            x = jnp.minimum(x[:n], x[n:])
        return jnp.min(x, axis=0, keepdims=True)

    m = jax.lax.fori_loop(0, NUM_NEIGHBORS - 1,
                          lambda _, m: masked_min(m), first_min())

    acc = None
    for d in range(3):
        diff = cpos[:, d:d + 1] - qT[d:d + 1, :]
        acc = diff * diff if acc is None else acc + diff * diff
    WT = jnp.where(p <= m, 1.0 / (acc + 1e-16), 0.0)
    wsum = jnp.sum(WT, axis=0, keepdims=True)

    qcT = jnp.dot(codesT_ref[...], WT,
                  preferred_element_type=jnp.float32) / wsum
    GT = jnp.dot(shT_ref[...], WT,
                 preferred_element_type=jnp.float32) / wsum

    vT = vT_ref[...]
    x = vT[0:1, :]
    y = vT[1:2, :]
    z = vT[2:3, :]
    xx, yy, zz = x * x, y * y, z * z
    shb = [
        jnp.full((1, QBLK), SH_C0, jnp.float32),
        -SH_C1 * y,
        SH_C1 * z,
        -SH_C1 * x,
        SH_C2[0] * (x * y),
        SH_C2[1] * (y * z),
        SH_C2[2] * (2.0 * zz - xx - yy),
        SH_C2[3] * (x * z),
        SH_C2[4] * (xx - yy),
    ]
    bidx = jax.lax.broadcasted_iota(jnp.int32, (SH_WIDTH, QBLK), 0) % SH_BASIS_DIM
    MT = jnp.zeros((SH_WIDTH, QBLK), jnp.float32)
    for b in range(SH_BASIS_DIM):
        MT = jnp.where(bidx == b, shb[b], MT)
    out_s_ref[...] = jnp.dot(sel_ref[...], GT * MT,
                             preferred_element_type=jnp.float32)
    out_c_ref[...] = qcT


def kernel(codes_position, codes, sh_codes, indices, query_points, viewdirs):
    idx0 = indices[0]
    cpos = codes_position[idx0]
    codesT = codes[idx0].T
    shT = sh_codes[idx0].T
    qT = query_points[0].T
    vT = viewdirs.T

    sel_np = np.zeros((CODE_DIM, SH_WIDTH), np.float32)
    sel_np[np.arange(SH_WIDTH) // SH_BASIS_DIM, np.arange(SH_WIDTH)] = 1.0
    sel = jnp.asarray(sel_np)

    grid = (NUM_POINTS // QBLK,)
    out_cT, out_sT = pl.pallas_call(
        _tc_body,
        grid=grid,
        in_specs=[
            pl.BlockSpec((3, QBLK), lambda i: (0, i)),
            pl.BlockSpec((3, QBLK), lambda i: (0, i)),
            pl.BlockSpec((NUM_CODES, 3), lambda i: (0, 0)),
            pl.BlockSpec((CODE_DIM, NUM_CODES), lambda i: (0, 0)),
            pl.BlockSpec((SH_WIDTH, NUM_CODES), lambda i: (0, 0)),
            pl.BlockSpec((CODE_DIM, SH_WIDTH), lambda i: (0, 0)),
        ],
        out_specs=[
            pl.BlockSpec((CODE_DIM, QBLK), lambda i: (0, i)),
            pl.BlockSpec((CODE_DIM, QBLK), lambda i: (0, i)),
        ],
        out_shape=[
            jax.ShapeDtypeStruct((CODE_DIM, NUM_POINTS), jnp.float32),
            jax.ShapeDtypeStruct((CODE_DIM, NUM_POINTS), jnp.float32),
        ],
    )(qT, vT, cpos, codesT, shT, sel)
    return (out_cT.T, out_sT.T)


# speedup vs baseline: 1.2209x; 1.2209x over previous
import jax
import jax.numpy as jnp
import numpy as np
from jax.experimental import pallas as pl

SH_C0 = 0.28209479177387814
SH_C1 = 0.4886025119029199
SH_C2 = [1.0925484305920792, -1.0925484305920792, 0.31539156525252005,
         -1.0925484305920792, 0.5462742152960396]

NUM_CODES = 4096
CODE_DIM = 32
NUM_NEIGHBORS = 16
SH_BASIS_DIM = 9
SH_WIDTH = CODE_DIM * SH_BASIS_DIM
NUM_POINTS = 8192
QBLK = 512
IDX_MASK = ~4095
INT_MAX = 2147483647


def _tc_body(qT_ref, vT_ref, cpos_ref, codesT_ref, shT_ref, sel_ref,
             out_c_ref, out_s_ref):
    qT = qT_ref[...]
    cpos = cpos_ref[...]
    mm = jnp.dot(cpos.astype(jnp.bfloat16), qT.astype(jnp.bfloat16),
                 preferred_element_type=jnp.float32)
    qq = jnp.sum(qT * qT, axis=0, keepdims=True)
    cc = jnp.sum(cpos * cpos, axis=1, keepdims=True)
    d2sel = qq + cc - 2.0 * mm
    iota = jax.lax.broadcasted_iota(jnp.int32, (NUM_CODES, QBLK), 0)
    p = jnp.bitwise_or(
        jnp.bitwise_and(jax.lax.bitcast_convert_type(d2sel, jnp.int32),
                        IDX_MASK),
        iota)

    m = jnp.min(p, axis=0, keepdims=True)

    def step(_, m):
        return jnp.min(jnp.where(p > m, p, INT_MAX), axis=0, keepdims=True)

    m = jax.lax.fori_loop(0, NUM_NEIGHBORS - 1, step, m)

    acc = None
    for d in range(3):
        diff = cpos[:, d:d + 1] - qT[d:d + 1, :]
        acc = diff * diff if acc is None else acc + diff * diff
    WT = jnp.where(p <= m, 1.0 / (acc + 1e-16), 0.0)
    wsum = jnp.sum(WT, axis=0, keepdims=True)

    qcT = jnp.dot(codesT_ref[...], WT,
                  preferred_element_type=jnp.float32) / wsum
    GT = jnp.dot(shT_ref[...], WT,
                 preferred_element_type=jnp.float32) / wsum

    vT = vT_ref[...]
    x = vT[0:1, :]
    y = vT[1:2, :]
    z = vT[2:3, :]
    xx, yy, zz = x * x, y * y, z * z
    shb = [
        jnp.full((1, QBLK), SH_C0, jnp.float32),
        -SH_C1 * y,
        SH_C1 * z,
        -SH_C1 * x,
        SH_C2[0] * (x * y),
        SH_C2[1] * (y * z),
        SH_C2[2] * (2.0 * zz - xx - yy),
        SH_C2[3] * (x * z),
        SH_C2[4] * (xx - yy),
    ]
    bidx = jax.lax.broadcasted_iota(jnp.int32, (SH_WIDTH, QBLK), 0) % SH_BASIS_DIM
    MT = jnp.zeros((SH_WIDTH, QBLK), jnp.float32)
    for b in range(SH_BASIS_DIM):
        MT = jnp.where(bidx == b, shb[b], MT)
    out_s_ref[...] = jnp.dot(sel_ref[...], GT * MT,
                             preferred_element_type=jnp.float32)
    out_c_ref[...] = qcT


def kernel(codes_position, codes, sh_codes, indices, query_points, viewdirs):
    idx0 = indices[0]
    cpos = codes_position[idx0]
    codesT = codes[idx0].T
    shT = sh_codes[idx0].T
    qT = query_points[0].T
    vT = viewdirs.T

    sel_np = np.zeros((CODE_DIM, SH_WIDTH), np.float32)
    sel_np[np.arange(SH_WIDTH) // SH_BASIS_DIM, np.arange(SH_WIDTH)] = 1.0
    sel = jnp.asarray(sel_np)

    grid = (NUM_POINTS // QBLK,)
    out_cT, out_sT = pl.pallas_call(
        _tc_body,
        grid=grid,
        in_specs=[
            pl.BlockSpec((3, QBLK), lambda i: (0, i)),
            pl.BlockSpec((3, QBLK), lambda i: (0, i)),
            pl.BlockSpec((NUM_CODES, 3), lambda i: (0, 0)),
            pl.BlockSpec((CODE_DIM, NUM_CODES), lambda i: (0, 0)),
            pl.BlockSpec((SH_WIDTH, NUM_CODES), lambda i: (0, 0)),
            pl.BlockSpec((CODE_DIM, SH_WIDTH), lambda i: (0, 0)),
        ],
        out_specs=[
            pl.BlockSpec((CODE_DIM, QBLK), lambda i: (0, i)),
            pl.BlockSpec((CODE_DIM, QBLK), lambda i: (0, i)),
        ],
        out_shape=[
            jax.ShapeDtypeStruct((CODE_DIM, NUM_POINTS), jnp.float32),
            jax.ShapeDtypeStruct((CODE_DIM, NUM_POINTS), jnp.float32),
        ],
    )(qT, vT, cpos, codesT, shT, sel)
    return (out_cT.T, out_sT.T)


# speedup vs baseline: 1.2709x; 1.0409x over previous
import jax
import jax.numpy as jnp
import numpy as np
from jax.experimental import pallas as pl

SH_C0 = 0.28209479177387814
SH_C1 = 0.4886025119029199
SH_C2 = [1.0925484305920792, -1.0925484305920792, 0.31539156525252005,
         -1.0925484305920792, 0.5462742152960396]

NUM_CODES = 4096
CODE_DIM = 32
NUM_NEIGHBORS = 16
SH_BASIS_DIM = 9
SH_WIDTH = CODE_DIM * SH_BASIS_DIM
NUM_POINTS = 8192
QBLK = 1024
IDX_MASK = ~4095
INT_MAX = 2147483647


def _tc_body(qT_ref, vT_ref, cpos_ref, codesT_ref, shT_ref, sel_ref,
             out_c_ref, out_s_ref):
    qT = qT_ref[...]
    cpos = cpos_ref[...]
    mm = jnp.dot(cpos.astype(jnp.bfloat16), qT.astype(jnp.bfloat16),
                 preferred_element_type=jnp.float32)
    qq = jnp.sum(qT * qT, axis=0, keepdims=True)
    cc = jnp.sum(cpos * cpos, axis=1, keepdims=True)
    d2sel = qq + cc - 2.0 * mm
    iota = jax.lax.broadcasted_iota(jnp.int32, (NUM_CODES, QBLK), 0)
    p = jnp.bitwise_or(
        jnp.bitwise_and(jax.lax.bitcast_convert_type(d2sel, jnp.int32),
                        IDX_MASK),
        iota)

    m = jnp.min(p, axis=0, keepdims=True)

    def step(_, m):
        return jnp.min(jnp.where(p > m, p, INT_MAX), axis=0, keepdims=True)

    m = jax.lax.fori_loop(0, NUM_NEIGHBORS - 1, step, m)

    acc = None
    for d in range(3):
        diff = cpos[:, d:d + 1] - qT[d:d + 1, :]
        acc = diff * diff if acc is None else acc + diff * diff
    WT = jnp.where(p <= m, 1.0 / (acc + 1e-16), 0.0)
    wsum = jnp.sum(WT, axis=0, keepdims=True)

    qcT = jnp.dot(codesT_ref[...], WT,
                  preferred_element_type=jnp.float32) / wsum
    GT = jnp.dot(shT_ref[...], WT,
                 preferred_element_type=jnp.float32) / wsum

    vT = vT_ref[...]
    x = vT[0:1, :]
    y = vT[1:2, :]
    z = vT[2:3, :]
    xx, yy, zz = x * x, y * y, z * z
    shb = [
        jnp.full((1, QBLK), SH_C0, jnp.float32),
        -SH_C1 * y,
        SH_C1 * z,
        -SH_C1 * x,
        SH_C2[0] * (x * y),
        SH_C2[1] * (y * z),
        SH_C2[2] * (2.0 * zz - xx - yy),
        SH_C2[3] * (x * z),
        SH_C2[4] * (xx - yy),
    ]
    bidx = jax.lax.broadcasted_iota(jnp.int32, (SH_WIDTH, QBLK), 0) % SH_BASIS_DIM
    MT = jnp.zeros((SH_WIDTH, QBLK), jnp.float32)
    for b in range(SH_BASIS_DIM):
        MT = jnp.where(bidx == b, shb[b], MT)
    out_s_ref[...] = jnp.dot(sel_ref[...], GT * MT,
                             preferred_element_type=jnp.float32)
    out_c_ref[...] = qcT


def kernel(codes_position, codes, sh_codes, indices, query_points, viewdirs):
    idx0 = indices[0]
    cpos = codes_position[idx0]
    codesT = codes[idx0].T
    shT = sh_codes[idx0].T
    qT = query_points[0].T
    vT = viewdirs.T

    sel_np = np.zeros((CODE_DIM, SH_WIDTH), np.float32)
    sel_np[np.arange(SH_WIDTH) // SH_BASIS_DIM, np.arange(SH_WIDTH)] = 1.0
    sel = jnp.asarray(sel_np)

    grid = (NUM_POINTS // QBLK,)
    out_cT, out_sT = pl.pallas_call(
        _tc_body,
        grid=grid,
        in_specs=[
            pl.BlockSpec((3, QBLK), lambda i: (0, i)),
            pl.BlockSpec((3, QBLK), lambda i: (0, i)),
            pl.BlockSpec((NUM_CODES, 3), lambda i: (0, 0)),
            pl.BlockSpec((CODE_DIM, NUM_CODES), lambda i: (0, 0)),
            pl.BlockSpec((SH_WIDTH, NUM_CODES), lambda i: (0, 0)),
            pl.BlockSpec((CODE_DIM, SH_WIDTH), lambda i: (0, 0)),
        ],
        out_specs=[
            pl.BlockSpec((CODE_DIM, QBLK), lambda i: (0, i)),
            pl.BlockSpec((CODE_DIM, QBLK), lambda i: (0, i)),
        ],
        out_shape=[
            jax.ShapeDtypeStruct((CODE_DIM, NUM_POINTS), jnp.float32),
            jax.ShapeDtypeStruct((CODE_DIM, NUM_POINTS), jnp.float32),
        ],
    )(qT, vT, cpos, codesT, shT, sel)
    return (out_cT.T, out_sT.T)
